# baseline (speedup 1.0000x reference)
import functools

import jax
import jax.numpy as jnp
from jax import lax
from jax.experimental import pallas as pl
from jax.experimental.pallas import tpu as pltpu
from jax.experimental.pallas import tpu_sc as plsc

_NC = 2
_NS = 16
_NW = _NC * _NS


def _table_body(emb_ref, wt_ref, b_ref, out_ref):
    out_ref[...] = (
        jnp.dot(emb_ref[...], wt_ref[...], preferred_element_type=jnp.float32)
        + b_ref[...]
    )


def _make_logits_table(emb, Wt, b2):
    V = emb.shape[0]
    Vp = Wt.shape[1]
    return pl.pallas_call(
        _table_body,
        out_shape=jax.ShapeDtypeStruct((V, Vp), jnp.float32),
    )(emb, Wt, b2)


def _make_gather(N, V, Vp, n_chunks, C):
    mesh = plsc.VectorSubcoreMesh(
        core_axis_name="c", subcore_axis_name="s",
        num_cores=_NC, num_subcores=_NS,
    )

    @functools.partial(
        pl.kernel,
        out_type=jax.ShapeDtypeStruct((N, V), jnp.float32),
        mesh=mesh,
        scratch_types=[
            pltpu.VMEM((n_chunks, C), jnp.int32),
            pltpu.VMEM((2, C, Vp), jnp.float32),
            pltpu.SemaphoreType.DMA,
            pltpu.SemaphoreType.DMA,
            pltpu.SemaphoreType.DMA,
            pltpu.SemaphoreType.DMA,
        ],
        compiler_params=pltpu.CompilerParams(use_tc_tiling_on_sc=False),
    )
    def gather(table_hbm, idx_hbm, out_hbm, idx_v, rows_v, g0, g1, o0, o1):
        wid = lax.axis_index("s") * _NC + lax.axis_index("c")
        base = wid * (n_chunks * C)
        gsem = (g0, g1)
        osem = (o0, o1)

        pltpu.sync_copy(idx_hbm.at[wid], idx_v)

        def start_gather(j, buf):
            pltpu.async_copy(
                table_hbm.at[idx_v.at[j]], rows_v.at[buf], gsem[buf]
            )

        def start_out(j, buf):
            pltpu.async_copy(
                rows_v.at[buf, :, pl.ds(0, V)],
                out_hbm.at[pl.ds(base + j * C, C)],
                osem[buf],
            )

        def wait_gather(buf):
            pltpu.make_async_copy(
                table_hbm.at[pl.ds(0, C)], rows_v.at[buf], gsem[buf]
            ).wait()

        def wait_out(buf):
            pltpu.make_async_copy(
                rows_v.at[buf, :, pl.ds(0, V)],
                out_hbm.at[pl.ds(base, C)],
                osem[buf],
            ).wait()

        start_gather(0, 0)
        start_gather(1, 1)

        def body(i, carry):
            g = 2 * i
            for buf in (0, 1):
                wait_gather(buf)
                start_out(g + buf, buf)
            for buf in (0, 1):
                wait_out(buf)
                start_gather(g + 2 + buf, buf)
            return carry

        lax.fori_loop(0, n_chunks // 2 - 1, body, 0)

        for buf in (0, 1):
            wait_gather(buf)
            start_out(n_chunks - 2 + buf, buf)
        for buf in (0, 1):
            wait_out(buf)

    return gather


def kernel(x, emb, W, b):
    V, D = emb.shape
    B, L = x.shape
    N = B * L

    rows_per_w = N // _NW
    C = 40
    n_chunks = rows_per_w // C

    Vp = (V + 127) // 128 * 128
    Wt = jnp.pad(W.T, ((0, 0), (0, Vp - V)))
    b2 = jnp.pad(b, (0, Vp - V)).reshape(1, Vp)
    table = _make_logits_table(emb, Wt, b2)

    xf = x.reshape(_NW, n_chunks, C).astype(jnp.int32)
    out = _make_gather(N, V, Vp, n_chunks, C)(table, xf)
    return out.reshape(B, L, V)
